# Initial kernel scaffold; baseline (speedup 1.0000x reference)
import jax
import jax.numpy as jnp
from jax.experimental import pallas as pl


def kernel(x_address, x_transaction, edge_index_a2t, edge_index_t2a, lin_w_addr, lin_b_addr, lin_w_tx, lin_b_tx, proj_w_addr_0, proj_b_addr_0, proj_w_tx_0, proj_b_tx_0, asrc_a2t_0, adst_a2t_0, asrc_t2a_0, adst_t2a_0, klin_w_0, klin_b_0, q_0, proj_w_addr_1, proj_b_addr_1, proj_w_tx_1, proj_b_tx_1, asrc_a2t_1, adst_a2t_1, asrc_t2a_1, adst_t2a_1, klin_w_1, klin_b_1, q_1):
    raise NotImplementedError("write your pallas kernel here")


# speedup vs baseline: 34.8770x; 34.8770x over previous
import functools

import jax
import jax.numpy as jnp
from jax import lax
from jax.experimental import pallas as pl
from jax.experimental.pallas import tpu as pltpu
from jax.experimental.pallas import tpu_sc as plsc

NEG_SLOPE = 0.2
H = 8
DH = 16
C = H * DH
EPS = 1e-16
_B = 80
_NSUB = 16


def _expand_a_fat(a):
    eye = jnp.eye(H, dtype=a.dtype)
    a8 = (a[:, :, None] * eye[:, None, :]).reshape(C, H)
    return jnp.concatenate(
        [a8, a8, jnp.zeros((C, C - 16), a.dtype)], axis=1)


def _scores_part(ha, ht, asA, adT, asT, adA,
                 sfa_o, dfa_o, sft_o, dft_o, ma2t_o, mt2a_o, mx, i):
    sfa = ha @ asA
    dfa = ht @ adT
    sft = ht @ asT
    dft = ha @ adA
    sfa_o[...] = sfa
    dfa_o[...] = dfa
    sft_o[...] = sft
    dft_o[...] = dft
    cur = jnp.concatenate(
        [jnp.max(sfa, axis=0, keepdims=True),
         jnp.max(dfa, axis=0, keepdims=True),
         jnp.max(sft, axis=0, keepdims=True),
         jnp.max(dft, axis=0, keepdims=True)], axis=0)

    @pl.when(i == 0)
    def _():
        mx[...] = cur

    @pl.when(i > 0)
    def _():
        mx[...] = jnp.maximum(mx[...], cur)

    m = mx[...]
    ua = m[0:1, :] + m[1:2, :]
    ut = m[2:3, :] + m[3:4, :]
    ma2t_o[...] = jnp.maximum(ua, NEG_SLOPE * ua)
    mt2a_o[...] = jnp.maximum(ut, NEG_SLOPE * ut)


def _prep0_body(xa_r, xt_r, wla, bla, wlt, blt, wpa, bpa, wpt, bpt,
                asA, adT, asT, adA,
                ha_o, ht_o, sfa_o, dfa_o, sft_o, dft_o, ma2t_o, mt2a_o, mx):
    i = pl.program_id(0)
    xa = jnp.maximum(xa_r[...] @ wla[...] + bla[...], 0.0)
    xt = jnp.maximum(xt_r[...] @ wlt[...] + blt[...], 0.0)
    ha = xa @ wpa[...] + bpa[...]
    ht = xt @ wpt[...] + bpt[...]
    ha_o[...] = ha
    ht_o[...] = ht
    _scores_part(ha, ht, asA[...], adT[...], asT[...], adA[...],
                 sfa_o, dfa_o, sft_o, dft_o, ma2t_o, mt2a_o, mx, i)


def _mid_body(acct_r, zt_r, acca_r, za_r,
              wpa, bpa, wpt, bpt, asA, adT, asT, adA,
              ha_o, ht_o, sfa_o, dfa_o, sft_o, dft_o, ma2t_o, mt2a_o, mx):
    i = pl.program_id(0)
    xt = jnp.maximum(acct_r[...] / (zt_r[...] + EPS), 0.0)
    xa = jnp.maximum(acca_r[...] / (za_r[...] + EPS), 0.0)
    ha = xa @ wpa[...] + bpa[...]
    ht = xt @ wpt[...] + bpt[...]
    ha_o[...] = ha
    ht_o[...] = ht
    _scores_part(ha, ht, asA[...], adT[...], asT[...], adA[...],
                 sfa_o, dfa_o, sft_o, dft_o, ma2t_o, mt2a_o, mx, i)


def _final_body(acca_r, za_r, out_o):
    out_o[...] = jnp.maximum(acca_r[...] / (za_r[...] + EPS), 0.0)


def _row_spec(rb, cols):
    return pl.BlockSpec((rb, cols), lambda i: (i, 0))


def _full_spec(shape):
    return pl.BlockSpec(shape, lambda i: tuple(0 for _ in shape))


def _prep_outs(n):
    f32 = jnp.float32
    return [jax.ShapeDtypeStruct((n, C), f32) for _ in range(6)] + [
        jax.ShapeDtypeStruct((1, C), f32),
        jax.ShapeDtypeStruct((1, C), f32),
    ]


def _prep_out_specs(rb):
    return [_row_spec(rb, C)] * 6 + [_full_spec((1, C)), _full_spec((1, C))]


def _tc_prep0(xa, xt, wla, bla, wlt, blt, wpa, bpa, wpt, bpt,
              asA, adT, asT, adA, rb=1000):
    n = xa.shape[0]
    return pl.pallas_call(
        _prep0_body,
        grid=(n // rb,),
        in_specs=[
            _row_spec(rb, C), _row_spec(rb, C),
            _full_spec((C, C)), _full_spec((1, C)),
            _full_spec((C, C)), _full_spec((1, C)),
            _full_spec((C, C)), _full_spec((1, C)),
            _full_spec((C, C)), _full_spec((1, C)),
            _full_spec((C, C)), _full_spec((C, C)),
            _full_spec((C, C)), _full_spec((C, C)),
        ],
        out_specs=_prep_out_specs(rb),
        out_shape=_prep_outs(n),
        scratch_shapes=[pltpu.VMEM((4, C), jnp.float32)],
    )(xa, xt, wla, bla, wlt, blt, wpa, bpa, wpt, bpt, asA, adT, asT, adA)


def _tc_mid(acct, zt, acca, za, wpa, bpa, wpt, bpt,
            asA, adT, asT, adA, rb=1024):
    n = acct.shape[0]
    return pl.pallas_call(
        _mid_body,
        grid=(n // rb,),
        in_specs=[
            _row_spec(rb, C), _row_spec(rb, C),
            _row_spec(rb, C), _row_spec(rb, C),
            _full_spec((C, C)), _full_spec((1, C)),
            _full_spec((C, C)), _full_spec((1, C)),
            _full_spec((C, C)), _full_spec((C, C)),
            _full_spec((C, C)), _full_spec((C, C)),
        ],
        out_specs=_prep_out_specs(rb),
        out_shape=_prep_outs(n),
        scratch_shapes=[pltpu.VMEM((4, C), jnp.float32)],
    )(acct, zt, acca, za, wpa, bpa, wpt, bpt, asA, adT, asT, adA)


def _tc_final(acca, za, rb=1024):
    n = acca.shape[0]
    return pl.pallas_call(
        _final_body,
        grid=(n // rb,),
        in_specs=[_row_spec(rb, C), _row_spec(rb, C)],
        out_specs=_row_spec(rb, C),
        out_shape=jax.ShapeDtypeStruct((n, C), jnp.float32),
    )(acca, za)


def _zero_acc(sid, rows, acc_sh, xbuf):
    zeros16 = jnp.zeros((16,), jnp.float32)

    def zz(r, carry):
        for h in range(H):
            xbuf[r, pl.ds(h * DH, DH)] = zeros16
        return carry

    lax.fori_loop(0, _B, zz, 0)
    r0 = sid * rows
    for k in range(rows // _B):
        pltpu.sync_copy(xbuf, acc_sh.at[pl.ds(r0 + k * _B, _B)])


def _sc_phase1(sid, X, SF, DF, M, eis, eid, e_out, nchunk, acc_sh,
               xbuf, sbuf, dbuf, ebuf, si_v, di_v, mv, sem1, sem2, sem3):
    pltpu.sync_copy(M.at[pl.ds(0, 16)], mv)
    m16 = mv[...]
    ebase = sid * (nchunk * _B)

    def chunk(c, carry):
        base = ebase + c * _B
        pltpu.sync_copy(eis.at[pl.ds(base, _B)], si_v)
        pltpu.sync_copy(eid.at[pl.ds(base, _B)], di_v)
        g1 = pltpu.async_copy(X.at[si_v], xbuf, sem1)
        g2 = pltpu.async_copy(SF.at[si_v], sbuf, sem2)
        g3 = pltpu.async_copy(DF.at[di_v], dbuf, sem3)
        g2.wait()
        g3.wait()
        g1.wait()

        def ebody(b, carry2):
            t = sbuf[b, pl.ds(0, 16)] + dbuf[b, pl.ds(0, 16)]
            a = jnp.maximum(t, NEG_SLOPE * t)
            e16 = jnp.exp(a - m16)
            ebuf[b, :] = e16
            for h in range(H):
                xbuf[b, pl.ds(h * DH, DH)] = xbuf[b, pl.ds(h * DH, DH)] * e16[h]
            return carry2

        lax.fori_loop(0, _B, ebody, 0)
        pltpu.sync_copy(ebuf, e_out.at[pl.ds(base, _B), :])
        pltpu.sync_copy(xbuf, acc_sh.at[di_v], add=True)
        return carry

    lax.fori_loop(0, nchunk, chunk, 0)


def _sc_phase2(sid, eid, e_in, nchunk, acc_sh, xbuf, ebuf, di_v, sem1):
    ebase = sid * (nchunk * _B)

    def chunk(c, carry):
        base = ebase + c * _B
        pltpu.sync_copy(eid.at[pl.ds(base, _B)], di_v)
        pltpu.sync_copy(e_in.at[pl.ds(base, _B), :], ebuf)

        ones16 = jnp.full((16,), 1.0, jnp.float32)

        def tbody(b, carry2):
            e16 = ebuf[b, :]
            for h in range(H):
                xbuf[b, pl.ds(h * DH, DH)] = ones16 * e16[h]
            return carry2

        lax.fori_loop(0, _B, tbody, 0)
        pltpu.sync_copy(xbuf, acc_sh.at[di_v], add=True)
        return carry

    lax.fori_loop(0, nchunk, chunk, 0)


def _sc_body(n, nchunk,
             ha, ht, sfa, dfa, sft, dft, ma2t, mt2a,
             si_a2t, di_a2t, si_t2a, di_t2a,
             acc_t, acc_a, z_t, z_a, e_a2t, e_t2a,
             acc_sh, xbuf, sbuf, dbuf, ebuf, si_v, di_v, mv,
             sem1, sem2, sem3):
    cid = lax.axis_index("c")
    sid = lax.axis_index("s")
    rows = n // _NSUB
    r0 = sid * rows

    _zero_acc(sid, rows, acc_sh, xbuf)
    plsc.subcore_barrier()

    @pl.when(cid == 0)
    def _():
        _sc_phase1(sid, ha, sfa, dfa, ma2t, si_a2t, di_a2t, e_a2t, nchunk,
                   acc_sh, xbuf, sbuf, dbuf, ebuf, si_v, di_v, mv,
                   sem1, sem2, sem3)

    @pl.when(cid == 1)
    def _():
        _sc_phase1(sid, ht, sft, dft, mt2a, si_t2a, di_t2a, e_t2a, nchunk,
                   acc_sh, xbuf, sbuf, dbuf, ebuf, si_v, di_v, mv,
                   sem1, sem2, sem3)

    plsc.subcore_barrier()

    @pl.when(cid == 0)
    def _():
        pltpu.sync_copy(acc_sh.at[pl.ds(r0, rows)], acc_t.at[pl.ds(r0, rows)])

    @pl.when(cid == 1)
    def _():
        pltpu.sync_copy(acc_sh.at[pl.ds(r0, rows)], acc_a.at[pl.ds(r0, rows)])

    plsc.subcore_barrier()
    _zero_acc(sid, rows, acc_sh, xbuf)
    plsc.subcore_barrier()

    @pl.when(cid == 0)
    def _():
        _sc_phase2(sid, di_a2t, e_a2t, nchunk, acc_sh, xbuf, ebuf, di_v, sem1)

    @pl.when(cid == 1)
    def _():
        _sc_phase2(sid, di_t2a, e_t2a, nchunk, acc_sh, xbuf, ebuf, di_v, sem1)

    plsc.subcore_barrier()

    @pl.when(cid == 0)
    def _():
        pltpu.sync_copy(acc_sh.at[pl.ds(r0, rows)], z_t.at[pl.ds(r0, rows)])

    @pl.when(cid == 1)
    def _():
        pltpu.sync_copy(acc_sh.at[pl.ds(r0, rows)], z_a.at[pl.ds(r0, rows)])


def _sc_edges(ha, ht, sfa, dfa, sft, dft, ma2t, mt2a, ei_a2t, ei_t2a, np_rows):
    e = ei_a2t.shape[1]
    si_a2t = ei_a2t[0].reshape(e)
    di_a2t = ei_a2t[1].reshape(e)
    si_t2a = ei_t2a[0].reshape(e)
    di_t2a = ei_t2a[1].reshape(e)
    assert e % (_NSUB * _B) == 0 and np_rows % (_NSUB * _B) == 0
    nchunk = e // (_NSUB * _B)
    f32, i32 = jnp.float32, jnp.int32
    mesh = plsc.VectorSubcoreMesh(core_axis_name="c", subcore_axis_name="s")
    kfn = pl.kernel(
        functools.partial(_sc_body, np_rows, nchunk),
        out_type=[
            jax.ShapeDtypeStruct((np_rows, C), f32),
            jax.ShapeDtypeStruct((np_rows, C), f32),
            jax.ShapeDtypeStruct((np_rows, C), f32),
            jax.ShapeDtypeStruct((np_rows, C), f32),
            jax.ShapeDtypeStruct((e, 16), f32),
            jax.ShapeDtypeStruct((e, 16), f32),
        ],
        mesh=mesh,
        scratch_types=[
            pltpu.VMEM_SHARED((np_rows, C), f32),
            pltpu.VMEM((_B, C), f32),
            pltpu.VMEM((_B, C), f32),
            pltpu.VMEM((_B, C), f32),
            pltpu.VMEM((_B, 16), f32),
            pltpu.VMEM((_B,), i32),
            pltpu.VMEM((_B,), i32),
            pltpu.VMEM((16,), f32),
            pltpu.SemaphoreType.DMA,
            pltpu.SemaphoreType.DMA,
            pltpu.SemaphoreType.DMA,
        ],
        compiler_params=pltpu.CompilerParams(needs_layout_passes=False),
    )
    acc_t, acc_a, z_t, z_a, _, _ = kfn(
        ha, ht, sfa, dfa, sft, dft, ma2t.reshape(C), mt2a.reshape(C),
        si_a2t, di_a2t, si_t2a, di_t2a)
    return acc_t, acc_a, z_t, z_a


def kernel(x_address, x_transaction, edge_index_a2t, edge_index_t2a,
           lin_w_addr, lin_b_addr, lin_w_tx, lin_b_tx,
           proj_w_addr_0, proj_b_addr_0, proj_w_tx_0, proj_b_tx_0,
           asrc_a2t_0, adst_a2t_0, asrc_t2a_0, adst_t2a_0,
           klin_w_0, klin_b_0, q_0,
           proj_w_addr_1, proj_b_addr_1, proj_w_tx_1, proj_b_tx_1,
           asrc_a2t_1, adst_a2t_1, asrc_t2a_1, adst_t2a_1,
           klin_w_1, klin_b_1, q_1):
    ei_a2t = edge_index_a2t.astype(jnp.int32)
    ei_t2a = edge_index_t2a.astype(jnp.int32)
    n = x_address.shape[0]
    np_rows = ((n + _NSUB * _B - 1) // (_NSUB * _B)) * (_NSUB * _B)

    def padn(x):
        return jnp.pad(x, ((0, np_rows - x.shape[0]), (0, 0)))

    ha, ht, sfa, dfa, sft, dft, m_a2t, m_t2a = _tc_prep0(
        x_address, x_transaction,
        lin_w_addr, lin_b_addr.reshape(1, C), lin_w_tx, lin_b_tx.reshape(1, C),
        proj_w_addr_0, proj_b_addr_0.reshape(1, C),
        proj_w_tx_0, proj_b_tx_0.reshape(1, C),
        _expand_a_fat(asrc_a2t_0), _expand_a_fat(adst_a2t_0),
        _expand_a_fat(asrc_t2a_0), _expand_a_fat(adst_t2a_0))

    acc_t, acc_a, z_t, z_a = _sc_edges(
        padn(ha), padn(ht), padn(sfa), padn(dfa), padn(sft), padn(dft),
        m_a2t, m_t2a, ei_a2t, ei_t2a, np_rows)

    ha, ht, sfa, dfa, sft, dft, m_a2t, m_t2a = _tc_mid(
        acc_t, z_t, acc_a, z_a,
        proj_w_addr_1, proj_b_addr_1.reshape(1, C),
        proj_w_tx_1, proj_b_tx_1.reshape(1, C),
        _expand_a_fat(asrc_a2t_1), _expand_a_fat(adst_a2t_1),
        _expand_a_fat(asrc_t2a_1), _expand_a_fat(adst_t2a_1))

    acc_t, acc_a, z_t, z_a = _sc_edges(
        ha, ht, sfa, dfa, sft, dft, m_a2t, m_t2a, ei_a2t, ei_t2a, np_rows)

    return _tc_final(acc_a, z_a)[:n]


# speedup vs baseline: 34.8881x; 1.0003x over previous
import functools

import jax
import jax.numpy as jnp
from jax import lax
from jax.experimental import pallas as pl
from jax.experimental.pallas import tpu as pltpu
from jax.experimental.pallas import tpu_sc as plsc

NEG_SLOPE = 0.2
H = 8
DH = 16
C = H * DH
EPS = 1e-16
_B = 80
_NSUB = 16


def _expand_a_fat(a):
    eye = jnp.eye(H, dtype=a.dtype)
    a8 = (a[:, :, None] * eye[:, None, :]).reshape(C, H)
    return jnp.concatenate(
        [a8, a8, jnp.zeros((C, C - 16), a.dtype)], axis=1)


def _scores_part(ha, ht, asA, adT, asT, adA,
                 sfa_o, dfa_o, sft_o, dft_o, ma2t_o, mt2a_o, mx, i):
    sfa = ha @ asA
    dfa = ht @ adT
    sft = ht @ asT
    dft = ha @ adA
    sfa_o[...] = sfa
    dfa_o[...] = dfa
    sft_o[...] = sft
    dft_o[...] = dft
    cur = jnp.concatenate(
        [jnp.max(sfa, axis=0, keepdims=True),
         jnp.max(dfa, axis=0, keepdims=True),
         jnp.max(sft, axis=0, keepdims=True),
         jnp.max(dft, axis=0, keepdims=True)], axis=0)

    @pl.when(i == 0)
    def _():
        mx[...] = cur

    @pl.when(i > 0)
    def _():
        mx[...] = jnp.maximum(mx[...], cur)

    m = mx[...]
    ua = m[0:1, :] + m[1:2, :]
    ut = m[2:3, :] + m[3:4, :]
    ma2t_o[...] = jnp.maximum(ua, NEG_SLOPE * ua)
    mt2a_o[...] = jnp.maximum(ut, NEG_SLOPE * ut)


def _prep0_body(xa_r, xt_r, wla, bla, wlt, blt, wpa, bpa, wpt, bpt,
                asA, adT, asT, adA,
                ha_o, ht_o, sfa_o, dfa_o, sft_o, dft_o, ma2t_o, mt2a_o, mx):
    i = pl.program_id(0)
    xa = jnp.maximum(xa_r[...] @ wla[...] + bla[...], 0.0)
    xt = jnp.maximum(xt_r[...] @ wlt[...] + blt[...], 0.0)
    ha = xa @ wpa[...] + bpa[...]
    ht = xt @ wpt[...] + bpt[...]
    ha_o[...] = ha
    ht_o[...] = ht
    _scores_part(ha, ht, asA[...], adT[...], asT[...], adA[...],
                 sfa_o, dfa_o, sft_o, dft_o, ma2t_o, mt2a_o, mx, i)


def _mid_body(acct_r, zt_r, acca_r, za_r,
              wpa, bpa, wpt, bpt, asA, adT, asT, adA,
              ha_o, ht_o, sfa_o, dfa_o, sft_o, dft_o, ma2t_o, mt2a_o, mx):
    i = pl.program_id(0)
    xt = jnp.maximum(acct_r[...] / (zt_r[...] + EPS), 0.0)
    xa = jnp.maximum(acca_r[...] / (za_r[...] + EPS), 0.0)
    ha = xa @ wpa[...] + bpa[...]
    ht = xt @ wpt[...] + bpt[...]
    ha_o[...] = ha
    ht_o[...] = ht
    _scores_part(ha, ht, asA[...], adT[...], asT[...], adA[...],
                 sfa_o, dfa_o, sft_o, dft_o, ma2t_o, mt2a_o, mx, i)


def _final_body(acca_r, za_r, out_o):
    out_o[...] = jnp.maximum(acca_r[...] / (za_r[...] + EPS), 0.0)


def _row_spec(rb, cols):
    return pl.BlockSpec((rb, cols), lambda i: (i, 0))


def _full_spec(shape):
    return pl.BlockSpec(shape, lambda i: tuple(0 for _ in shape))


def _prep_outs(n):
    f32 = jnp.float32
    return [jax.ShapeDtypeStruct((n, C), f32) for _ in range(6)] + [
        jax.ShapeDtypeStruct((1, C), f32),
        jax.ShapeDtypeStruct((1, C), f32),
    ]


def _prep_out_specs(rb):
    return [_row_spec(rb, C)] * 6 + [_full_spec((1, C)), _full_spec((1, C))]


def _tc_prep0(xa, xt, wla, bla, wlt, blt, wpa, bpa, wpt, bpt,
              asA, adT, asT, adA, rb=1000):
    n = xa.shape[0]
    return pl.pallas_call(
        _prep0_body,
        grid=(n // rb,),
        in_specs=[
            _row_spec(rb, C), _row_spec(rb, C),
            _full_spec((C, C)), _full_spec((1, C)),
            _full_spec((C, C)), _full_spec((1, C)),
            _full_spec((C, C)), _full_spec((1, C)),
            _full_spec((C, C)), _full_spec((1, C)),
            _full_spec((C, C)), _full_spec((C, C)),
            _full_spec((C, C)), _full_spec((C, C)),
        ],
        out_specs=_prep_out_specs(rb),
        out_shape=_prep_outs(n),
        scratch_shapes=[pltpu.VMEM((4, C), jnp.float32)],
    )(xa, xt, wla, bla, wlt, blt, wpa, bpa, wpt, bpt, asA, adT, asT, adA)


def _tc_mid(acct, zt, acca, za, wpa, bpa, wpt, bpt,
            asA, adT, asT, adA, rb=1024):
    n = acct.shape[0]
    return pl.pallas_call(
        _mid_body,
        grid=(n // rb,),
        in_specs=[
            _row_spec(rb, C), _row_spec(rb, C),
            _row_spec(rb, C), _row_spec(rb, C),
            _full_spec((C, C)), _full_spec((1, C)),
            _full_spec((C, C)), _full_spec((1, C)),
            _full_spec((C, C)), _full_spec((C, C)),
            _full_spec((C, C)), _full_spec((C, C)),
        ],
        out_specs=_prep_out_specs(rb),
        out_shape=_prep_outs(n),
        scratch_shapes=[pltpu.VMEM((4, C), jnp.float32)],
    )(acct, zt, acca, za, wpa, bpa, wpt, bpt, asA, adT, asT, adA)


def _tc_final(acca, za, rb=1024):
    n = acca.shape[0]
    return pl.pallas_call(
        _final_body,
        grid=(n // rb,),
        in_specs=[_row_spec(rb, C), _row_spec(rb, C)],
        out_specs=_row_spec(rb, C),
        out_shape=jax.ShapeDtypeStruct((n, C), jnp.float32),
    )(acca, za)


def _zero_acc(sid, rows, acc_sh, xbuf):
    zeros16 = jnp.zeros((16,), jnp.float32)

    def zz(r, carry):
        for h in range(H):
            xbuf[r, pl.ds(h * DH, DH)] = zeros16
        return carry

    lax.fori_loop(0, _B, zz, 0)
    r0 = sid * rows
    for k in range(rows // _B):
        pltpu.sync_copy(xbuf, acc_sh.at[pl.ds(r0 + k * _B, _B)])


def _sc_phase1(sid, X, SF, DF, M, eis, eid, e_out, nchunk, acc_sh,
               xbuf, sbuf, dbuf, ebuf, si_v, di_v, mv, sem1, sem2, sem3):
    pltpu.sync_copy(M.at[pl.ds(0, 16)], mv)
    m16 = mv[...]
    ebase = sid * (nchunk * _B)

    def chunk(c, carry):
        base = ebase + c * _B
        pltpu.sync_copy(eis.at[pl.ds(base, _B)], si_v)
        pltpu.sync_copy(eid.at[pl.ds(base, _B)], di_v)
        g1 = pltpu.async_copy(X.at[si_v], xbuf, sem1)
        g2 = pltpu.async_copy(SF.at[si_v], sbuf, sem2)
        g3 = pltpu.async_copy(DF.at[di_v], dbuf, sem3)
        g2.wait()
        g3.wait()
        g1.wait()

        def ebody(b, carry2):
            t = sbuf[b, pl.ds(0, 16)] + dbuf[b, pl.ds(0, 16)]
            a = jnp.maximum(t, NEG_SLOPE * t)
            e16 = jnp.exp(a - m16)
            ebuf[b, :] = e16
            for h in range(H):
                xbuf[b, pl.ds(h * DH, DH)] = xbuf[b, pl.ds(h * DH, DH)] * e16[h]
            return carry2

        lax.fori_loop(0, _B, ebody, 0)
        pltpu.sync_copy(ebuf, e_out.at[pl.ds(base, _B), :])
        pltpu.sync_copy(xbuf, acc_sh.at[di_v], add=True)
        return carry

    lax.fori_loop(0, nchunk, chunk, 0)


def _sc_phase2(sid, eid, e_in, nchunk, acc_sh, xbuf, ebuf, di_v, sem1):
    ebase = sid * (nchunk * _B)

    def chunk(c, carry):
        base = ebase + c * _B
        pltpu.sync_copy(eid.at[pl.ds(base, _B)], di_v)
        pltpu.sync_copy(e_in.at[pl.ds(base, _B), :], ebuf)

        ones16 = jnp.full((16,), 1.0, jnp.float32)

        def tbody(b, carry2):
            e16 = ebuf[b, :]
            for h in range(H):
                xbuf[b, pl.ds(h * DH, DH)] = ones16 * e16[h]
            return carry2

        lax.fori_loop(0, _B, tbody, 0)
        pltpu.sync_copy(xbuf, acc_sh.at[di_v], add=True)
        return carry

    lax.fori_loop(0, nchunk, chunk, 0)


def _sc_body(n, nchunk,
             ha, ht, sfa, dfa, sft, dft, ma2t, mt2a,
             si_a2t, di_a2t, si_t2a, di_t2a,
             acc_t, acc_a, z_t, z_a, e_a2t, e_t2a,
             acc_sh, xbuf, sbuf, dbuf, ebuf, si_v, di_v, mv,
             sem1, sem2, sem3):
    cid = lax.axis_index("c")
    sid = lax.axis_index("s")
    rows = n // _NSUB
    r0 = sid * rows

    _zero_acc(sid, rows, acc_sh, xbuf)
    plsc.subcore_barrier()

    @pl.when(cid == 0)
    def _():
        _sc_phase1(sid, ha, sfa, dfa, ma2t, si_a2t, di_a2t, e_a2t, nchunk,
                   acc_sh, xbuf, sbuf, dbuf, ebuf, si_v, di_v, mv,
                   sem1, sem2, sem3)

    @pl.when(cid == 1)
    def _():
        _sc_phase1(sid, ht, sft, dft, mt2a, si_t2a, di_t2a, e_t2a, nchunk,
                   acc_sh, xbuf, sbuf, dbuf, ebuf, si_v, di_v, mv,
                   sem1, sem2, sem3)

    plsc.subcore_barrier()

    @pl.when(cid == 0)
    def _():
        pltpu.sync_copy(acc_sh.at[pl.ds(r0, rows)], acc_t.at[pl.ds(r0, rows)])

    @pl.when(cid == 1)
    def _():
        pltpu.sync_copy(acc_sh.at[pl.ds(r0, rows)], acc_a.at[pl.ds(r0, rows)])

    plsc.subcore_barrier()
    _zero_acc(sid, rows, acc_sh, xbuf)
    plsc.subcore_barrier()

    @pl.when(cid == 0)
    def _():
        _sc_phase2(sid, di_a2t, e_a2t, nchunk, acc_sh, xbuf, ebuf, di_v, sem1)

    @pl.when(cid == 1)
    def _():
        _sc_phase2(sid, di_t2a, e_t2a, nchunk, acc_sh, xbuf, ebuf, di_v, sem1)

    plsc.subcore_barrier()

    @pl.when(cid == 0)
    def _():
        pltpu.sync_copy(acc_sh.at[pl.ds(r0, rows)], z_t.at[pl.ds(r0, rows)])

    @pl.when(cid == 1)
    def _():
        pltpu.sync_copy(acc_sh.at[pl.ds(r0, rows)], z_a.at[pl.ds(r0, rows)])


def _sc_edges(ha, ht, sfa, dfa, sft, dft, ma2t, mt2a, ei_a2t, ei_t2a, np_rows):
    e0 = ei_a2t.shape[1]
    e = ((e0 + _NSUB * _B - 1) // (_NSUB * _B)) * (_NSUB * _B)

    def padi(row, fill):
        return jnp.concatenate(
            [row.reshape(e0), jnp.full((e - e0,), fill, jnp.int32)])

    si_a2t = padi(ei_a2t[0], 0)
    di_a2t = padi(ei_a2t[1], np_rows - 1)
    si_t2a = padi(ei_t2a[0], 0)
    di_t2a = padi(ei_t2a[1], np_rows - 1)
    assert np_rows % (_NSUB * _B) == 0
    nchunk = e // (_NSUB * _B)
    f32, i32 = jnp.float32, jnp.int32
    mesh = plsc.VectorSubcoreMesh(core_axis_name="c", subcore_axis_name="s")
    kfn = pl.kernel(
        functools.partial(_sc_body, np_rows, nchunk),
        out_type=[
            jax.ShapeDtypeStruct((np_rows, C), f32),
            jax.ShapeDtypeStruct((np_rows, C), f32),
            jax.ShapeDtypeStruct((np_rows, C), f32),
            jax.ShapeDtypeStruct((np_rows, C), f32),
            jax.ShapeDtypeStruct((e, 16), f32),
            jax.ShapeDtypeStruct((e, 16), f32),
        ],
        mesh=mesh,
        scratch_types=[
            pltpu.VMEM_SHARED((np_rows, C), f32),
            pltpu.VMEM((_B, C), f32),
            pltpu.VMEM((_B, C), f32),
            pltpu.VMEM((_B, C), f32),
            pltpu.VMEM((_B, 16), f32),
            pltpu.VMEM((_B,), i32),
            pltpu.VMEM((_B,), i32),
            pltpu.VMEM((16,), f32),
            pltpu.SemaphoreType.DMA,
            pltpu.SemaphoreType.DMA,
            pltpu.SemaphoreType.DMA,
        ],
        compiler_params=pltpu.CompilerParams(needs_layout_passes=False),
    )
    acc_t, acc_a, z_t, z_a, _, _ = kfn(
        ha, ht, sfa, dfa, sft, dft, ma2t.reshape(C), mt2a.reshape(C),
        si_a2t, di_a2t, si_t2a, di_t2a)
    return acc_t, acc_a, z_t, z_a


def kernel(x_address, x_transaction, edge_index_a2t, edge_index_t2a,
           lin_w_addr, lin_b_addr, lin_w_tx, lin_b_tx,
           proj_w_addr_0, proj_b_addr_0, proj_w_tx_0, proj_b_tx_0,
           asrc_a2t_0, adst_a2t_0, asrc_t2a_0, adst_t2a_0,
           klin_w_0, klin_b_0, q_0,
           proj_w_addr_1, proj_b_addr_1, proj_w_tx_1, proj_b_tx_1,
           asrc_a2t_1, adst_a2t_1, asrc_t2a_1, adst_t2a_1,
           klin_w_1, klin_b_1, q_1):
    ei_a2t = edge_index_a2t.astype(jnp.int32)
    ei_t2a = edge_index_t2a.astype(jnp.int32)
    n = x_address.shape[0]
    np_rows = ((n + _NSUB * _B - 1) // (_NSUB * _B)) * (_NSUB * _B)

    def padn(x):
        return jnp.pad(x, ((0, np_rows - x.shape[0]), (0, 0)))

    ha, ht, sfa, dfa, sft, dft, m_a2t, m_t2a = _tc_prep0(
        x_address, x_transaction,
        lin_w_addr, lin_b_addr.reshape(1, C), lin_w_tx, lin_b_tx.reshape(1, C),
        proj_w_addr_0, proj_b_addr_0.reshape(1, C),
        proj_w_tx_0, proj_b_tx_0.reshape(1, C),
        _expand_a_fat(asrc_a2t_0), _expand_a_fat(adst_a2t_0),
        _expand_a_fat(asrc_t2a_0), _expand_a_fat(adst_t2a_0))

    acc_t, acc_a, z_t, z_a = _sc_edges(
        padn(ha), padn(ht), padn(sfa), padn(dfa), padn(sft), padn(dft),
        m_a2t, m_t2a, ei_a2t, ei_t2a, np_rows)

    ha, ht, sfa, dfa, sft, dft, m_a2t, m_t2a = _tc_mid(
        acc_t, z_t, acc_a, z_a,
        proj_w_addr_1, proj_b_addr_1.reshape(1, C),
        proj_w_tx_1, proj_b_tx_1.reshape(1, C),
        _expand_a_fat(asrc_a2t_1), _expand_a_fat(adst_a2t_1),
        _expand_a_fat(asrc_t2a_1), _expand_a_fat(adst_t2a_1))

    acc_t, acc_a, z_t, z_a = _sc_edges(
        ha, ht, sfa, dfa, sft, dft, m_a2t, m_t2a, ei_a2t, ei_t2a, np_rows)

    return _tc_final(acc_a, z_a)[:n]
